# baseline (speedup 1.0000x reference)
import functools

import jax
import jax.numpy as jnp
from jax import lax
from jax.experimental import pallas as pl
from jax.experimental.pallas import tpu as pltpu
from jax.experimental.pallas import tpu_sc as plsc

_KNN = 14
_KPAD = 16
_BQ = 64
_L = 128
_DPAD = 128
_NF = 5


def _topk_body(q_ref, k_ref, qsq_ref, ksq_ref, w_ref, i_ref, d_ref):
    BQ = q_ref.shape[0]
    K = k_ref.shape[0]
    C = K // _L
    q = q_ref[...]
    ks = k_ref[...]
    qk = jax.lax.dot_general(
        q, ks, (((1,), (1,)), ((), ())),
        preferred_element_type=jnp.float32)
    dists = qsq_ref[...] + ksq_ref[...] - 2.0 * qk
    d_ref[...] = dists

    inf = jnp.float32(jnp.inf)
    big = jnp.float32(1e9)
    lane = jax.lax.broadcasted_iota(jnp.int32, (BQ, _L), 1).astype(jnp.float32)

    def sweep(c, ms):
        m1, m2, m3, m4, c1, c2, c3, c4 = ms
        v = d_ref[:, pl.ds(c * _L, _L)]
        cc = lane + jnp.float32(_L) * c
        lt = v < m1
        s = jnp.where(lt, m1, v)
        cs = jnp.where(lt, c1, cc)
        m1 = jnp.where(lt, v, m1)
        c1 = jnp.where(lt, cc, c1)
        lt = s < m2
        t = jnp.where(lt, m2, s)
        ct = jnp.where(lt, c2, cs)
        m2 = jnp.where(lt, s, m2)
        c2 = jnp.where(lt, cs, c2)
        lt = t < m3
        u = jnp.where(lt, m3, t)
        cu = jnp.where(lt, c3, ct)
        m3 = jnp.where(lt, t, m3)
        c3 = jnp.where(lt, ct, c3)
        lt = u < m4
        m4 = jnp.where(lt, u, m4)
        c4 = jnp.where(lt, cu, c4)
        return (m1, m2, m3, m4, c1, c2, c3, c4)

    init = tuple(jnp.full((BQ, _L), inf, jnp.float32) for _ in range(4)) + \
        tuple(jnp.zeros((BQ, _L), jnp.float32) for _ in range(4))
    res = jax.lax.fori_loop(0, C, sweep, init)
    cand = jnp.concatenate(res[:4], axis=1)
    cidx = jnp.concatenate(res[4:], axis=1)

    vs, ids = [], []
    cur = cand
    for i in range(_KNN):
        m = jnp.min(cur, axis=1, keepdims=True)
        eq = cur == m
        ii = jnp.min(jnp.where(eq, cidx, big), axis=1, keepdims=True)
        vs.append(m)
        ids.append(ii)
        cur = jnp.where(eq, inf, cur)
    t_cand = vs[-1]

    d2 = d_ref[...]
    cnt = jnp.sum(jnp.where(d2 < t_cand, 1.0, 0.0), axis=1, keepdims=True)
    bad = jnp.any(cnt > 13.5)

    vals0 = jnp.concatenate(vs, axis=1)
    idxs0 = jnp.concatenate(ids, axis=1)

    zc = jnp.zeros((BQ, _KPAD - _KNN), jnp.float32)
    vals0p = jnp.concatenate([vals0, zc], axis=1)
    idxs0p = jnp.concatenate([idxs0, zc], axis=1)

    lane16 = jax.lax.broadcasted_iota(jnp.int32, (BQ, _KPAD), 1)

    def full_extract(_):
        def fe_body(i, carry):
            va, ia = carry
            curf = d_ref[...]
            mf = jnp.min(curf, axis=1, keepdims=True)
            eqf = curf == mf
            iota = jax.lax.broadcasted_iota(
                jnp.int32, (BQ, K), 1).astype(jnp.float32)
            ii = jnp.min(jnp.where(eqf, iota, big), axis=1, keepdims=True)
            d_ref[...] = jnp.where(eqf, inf, curf)
            oh = (lane16 == i).astype(jnp.float32)
            return va + mf * oh, ia + ii * oh
        z = jnp.zeros((BQ, _KPAD), jnp.float32)
        return jax.lax.fori_loop(0, _KNN, fe_body, (z, z))

    vals, idxs = jax.lax.cond(
        bad, full_extract, lambda _: (vals0p, idxs0p), 0)

    vals = vals[:, :_KNN]
    idxs = idxs[:, :_KNN]
    v1 = vals[:, :1]
    w = jnp.exp(v1 - vals)
    w = w / jnp.sum(w, axis=1, keepdims=True)
    zpad = jnp.zeros((BQ, _KPAD - _KNN), jnp.float32)
    wn = jnp.concatenate([w, zpad], axis=1)
    w_ref[...] = jnp.broadcast_to(
        wn[:, :, None], (BQ, _KPAD, 16)).reshape(BQ * _KPAD, 16)
    i_ref[...] = jnp.concatenate([idxs, zpad], axis=1).astype(jnp.int32)


def _topk_tc(queries, keys):
    Q, d = queries.shape
    K = keys.shape[0]
    q_sq = jnp.sum(queries * queries, axis=1, keepdims=True)
    k_sq = jnp.sum(keys * keys, axis=1)[None, :]
    return pl.pallas_call(
        _topk_body,
        grid=(Q // _BQ,),
        in_specs=[
            pl.BlockSpec((_BQ, d), lambda i: (i, 0)),
            pl.BlockSpec((K, d), lambda i: (0, 0)),
            pl.BlockSpec((_BQ, 1), lambda i: (i, 0)),
            pl.BlockSpec((1, K), lambda i: (0, 0)),
        ],
        out_specs=[
            pl.BlockSpec((_BQ * _KPAD, 16), lambda i: (i, 0)),
            pl.BlockSpec((_BQ, _KPAD), lambda i: (i, 0)),
        ],
        out_shape=[
            jax.ShapeDtypeStruct((Q * _KPAD, 16), jnp.float32),
            jax.ShapeDtypeStruct((Q, _KPAD), jnp.int32),
        ],
        scratch_shapes=[
            pltpu.VMEM((_BQ, K), jnp.float32),
        ],
    )(queries, keys, q_sq, k_sq)


def _gather_sc(keys_pad, idx_flat, w_flat, Q):
    info = plsc.get_sparse_core_info()
    NC, NS = info.num_cores, info.num_subcores
    NW = NC * NS
    per_w = Q // NW
    G = 8
    n_chunks = per_w // G
    mesh = plsc.VectorSubcoreMesh(core_axis_name="c", subcore_axis_name="s")

    @functools.partial(
        pl.kernel, mesh=mesh,
        out_type=jax.ShapeDtypeStruct((Q, _DPAD), jnp.float32),
        scratch_types=[
            pltpu.VMEM((G * _KPAD,), jnp.int32),
            pltpu.VMEM((G * _KPAD, 16), jnp.float32),
            pltpu.VMEM((G * _KPAD, _DPAD), jnp.float32),
            pltpu.VMEM((G, _DPAD), jnp.float32),
            pltpu.SemaphoreType.DMA,
        ],
    )
    def sc_kernel(keys_hbm, idx_hbm, w_hbm, out_hbm, idx_v, w_v, rows_v, out_v, sem):
        wid = lax.axis_index("s") * NC + lax.axis_index("c")
        qbase = wid * per_w

        def chunk(ci, carry):
            qoff = qbase + ci * G
            ioff = qoff * _KPAD
            pltpu.sync_copy(idx_hbm.at[pl.ds(ioff, G * _KPAD)], idx_v)
            pltpu.sync_copy(w_hbm.at[pl.ds(ioff, G * _KPAD)], w_v)
            pltpu.async_copy(keys_hbm.at[idx_v], rows_v, sem).wait()
            for q in range(G):
                accs = [jnp.zeros((16,), jnp.float32) for _ in range(_NF)]
                for i in range(_KNN):
                    r = q * _KPAD + i
                    wv = w_v[r, :]
                    for j in range(_NF):
                        accs[j] = accs[j] + wv * rows_v[r, pl.ds(j * 16, 16)]
                for j in range(_NF):
                    out_v[q, pl.ds(j * 16, 16)] = accs[j]
            pltpu.sync_copy(out_v, out_hbm.at[pl.ds(qoff, G)])
            return carry

        jax.lax.fori_loop(0, n_chunks, chunk, 0)

    return sc_kernel(keys_pad, idx_flat, w_flat)


def kernel(queries, keys, k):
    Q, d = queries.shape
    w, idx = _topk_tc(queries, keys)
    keys_pad = jnp.pad(keys, ((0, 0), (0, _DPAD - d)))
    out = _gather_sc(keys_pad, idx.reshape(-1), w, Q)
    return out[:, :d]


# speedup vs baseline: 2.5951x; 2.5951x over previous
import jax
import jax.numpy as jnp
from jax.experimental import pallas as pl
from jax.experimental.pallas import tpu as pltpu

_KNN = 14
_BQ = 128
_L = 128


def _nld_body(q_ref, k_ref, qsq_ref, ksq_ref, o_ref, d_ref):
    BQ = q_ref.shape[0]
    K = k_ref.shape[0]
    C = K // _L
    q = q_ref[...]
    ks = k_ref[...]
    qk = jax.lax.dot_general(
        q, ks, (((1,), (1,)), ((), ())),
        preferred_element_type=jnp.float32)
    dists = qsq_ref[...] + ksq_ref[...] - 2.0 * qk
    d_ref[...] = dists

    inf = jnp.float32(jnp.inf)

    def sweep(c, ms):
        m1, m2, m3, m4 = ms
        v = d_ref[:, pl.ds(c * _L, _L)]
        s = jnp.maximum(m1, v)
        m1 = jnp.minimum(m1, v)
        t = jnp.maximum(m2, s)
        m2 = jnp.minimum(m2, s)
        u = jnp.maximum(m3, t)
        m3 = jnp.minimum(m3, t)
        m4 = jnp.minimum(m4, u)
        return (m1, m2, m3, m4)

    init = tuple(jnp.full((BQ, _L), inf, jnp.float32) for _ in range(4))
    m1, m2, m3, m4 = jax.lax.fori_loop(0, C, sweep, init)
    cand = jnp.concatenate([m1, m2, m3, m4], axis=1)

    v1 = None
    t_cand = None
    cur = cand
    for i in range(_KNN):
        m = jnp.min(cur, axis=1, keepdims=True)
        if i == 0:
            v1 = m
        t_cand = m
        cur = jnp.where(cur == m, inf, cur)

    d2 = d_ref[...]
    cnt = jnp.sum(jnp.where(d2 < t_cand, 1.0, 0.0), axis=1, keepdims=True)
    bad = jnp.any(cnt > 13.5)

    def full_extract(_):
        def fe_body(i, tt):
            curf = d_ref[...]
            mf = jnp.min(curf, axis=1, keepdims=True)
            d_ref[...] = jnp.where(curf == mf, inf, curf)
            return mf
        return jax.lax.fori_loop(0, _KNN, fe_body, t_cand)

    t_fin = jax.lax.cond(bad, full_extract, lambda _: t_cand, 0)

    w = jnp.where(d2 <= t_fin, jnp.exp(v1 - d2), 0.0)
    denom = jnp.sum(w, axis=1, keepdims=True)
    num = jax.lax.dot_general(
        w.astype(jnp.bfloat16), ks.astype(jnp.bfloat16),
        (((1,), (0,)), ((), ())),
        preferred_element_type=jnp.float32)
    o_ref[...] = num / denom


def kernel(queries, keys, k):
    Q, d = queries.shape
    K = keys.shape[0]
    q_sq = jnp.sum(queries * queries, axis=1, keepdims=True)
    k_sq = jnp.sum(keys * keys, axis=1)[None, :]
    out = pl.pallas_call(
        _nld_body,
        grid=(Q // _BQ,),
        in_specs=[
            pl.BlockSpec((_BQ, d), lambda i: (i, 0)),
            pl.BlockSpec((K, d), lambda i: (0, 0)),
            pl.BlockSpec((_BQ, 1), lambda i: (i, 0)),
            pl.BlockSpec((1, K), lambda i: (0, 0)),
        ],
        out_specs=pl.BlockSpec((_BQ, d), lambda i: (i, 0)),
        out_shape=jax.ShapeDtypeStruct((Q, d), jnp.float32),
        scratch_shapes=[
            pltpu.VMEM((_BQ, K), jnp.float32),
        ],
    )(queries, keys, q_sq, k_sq)
    return out
